# baseline (speedup 1.0000x reference)
import functools

import jax
import jax.numpy as jnp
import numpy as np
from jax.experimental import pallas as pl
from jax.experimental.pallas import tpu as pltpu

_VMEM = 64 * 1024 * 1024


def _triangle_schedule(t):
    blocks = [(i, j) for i in range(t) for j in range(i + 1)]
    core0 = blocks[0::2]
    core1 = blocks[1::2]
    m = max(len(core0), len(core1))

    def pack(lst):
        act = [1] * len(lst) + [0] * (m - len(lst))
        filler = lst[-1] if lst else (0, 0)
        lst = lst + [filler] * (m - len(lst))
        return lst, act

    core0, act0 = pack(core0)
    core1, act1 = pack(core1)
    i_map = np.array([[b[0] for b in core0], [b[0] for b in core1]], np.int32)
    j_map = np.array([[b[1] for b in core0], [b[1] for b in core1]], np.int32)
    b_map = np.array([[b[0] * (b[0] + 1) // 2 + b[1] for b in core0],
                      [b[0] * (b[0] + 1) // 2 + b[1] for b in core1]],
                     np.int32)
    act = np.array([act0, act1], np.int32)
    return i_map, j_map, b_map, act, m


def _tri_dots(acc_ref, pk_ref, p_ref, i, j, active, tm):
    cm = min(tm, 1024)

    @pl.when(active == 1)
    def _do():
        pj = p_ref[pl.ds(j * tm, tm), :]
        for r in range(tm // cm):
            acc_ref[pl.ds(i * tm + r * cm, cm), :] += jax.lax.dot_general(
                pk_ref[0, pl.ds(r * cm, cm), :], pj, (((1,), (0,)), ((), ())),
                preferred_element_type=jnp.float32)

        @pl.when(i != j)
        def _sym():
            pi = p_ref[pl.ds(i * tm, tm), :]
            for s in range(tm // cm):
                acc_ref[pl.ds(j * tm + s * cm, cm), :] += jax.lax.dot_general(
                    pk_ref[0, :, pl.ds(s * cm, cm)], pi,
                    (((0,), (0,)), ((), ())),
                    preferred_element_type=jnp.float32)


def _sym_acc_kernel(i_ref, j_ref, b_ref, act_ref, a_ref, p_ref, out_ref,
                    a_bf_ref, acc_ref, sem, *, tm):
    c = pl.program_id(0)
    m = pl.program_id(1)
    nm = pl.num_programs(1)
    i = i_ref[c, m]
    j = j_ref[c, m]
    active = act_ref[c, m]

    @pl.when(m == 0)
    def _zero():
        acc_ref[...] = jnp.zeros_like(acc_ref)

    a_bf_ref[0] = a_ref[...].astype(jnp.bfloat16)
    _tri_dots(acc_ref, a_bf_ref, p_ref, i, j, active, tm)

    @pl.when(m == nm - 1)
    def _flush():
        copy = pltpu.make_async_copy(acc_ref, out_ref.at[c], sem)
        copy.start()
        copy.wait()


def _sym_matmul(a_hat, p, tm):
    n = a_hat.shape[0]
    k = p.shape[1]
    t = n // tm
    i_map, j_map, b_map, act, m = _triangle_schedule(t)
    tri_blocks = t * (t + 1) // 2

    grid_spec = pltpu.PrefetchScalarGridSpec(
        num_scalar_prefetch=4,
        grid=(2, m),
        in_specs=[
            pl.BlockSpec((tm, tm),
                         lambda c, mm, i_m, j_m, b_m, a_m: (i_m[c, mm],
                                                            j_m[c, mm])),
            pl.BlockSpec((n, k), lambda c, mm, i_m, j_m, b_m, a_m: (0, 0)),
        ],
        out_specs=pl.BlockSpec(memory_space=pl.ANY),
        scratch_shapes=[pltpu.VMEM((1, tm, tm), jnp.bfloat16),
                        pltpu.VMEM((n, k), jnp.float32),
                        pltpu.SemaphoreType.DMA],
    )
    flops = 2 * (2 * tri_blocks - t) * tm * tm * k
    bytes_accessed = 4 * tri_blocks * tm * tm + 2 * n * k + 2 * 4 * n * k
    return pl.pallas_call(
        functools.partial(_sym_acc_kernel, tm=tm),
        grid_spec=grid_spec,
        out_shape=jax.ShapeDtypeStruct((2, n, k), jnp.float32),
        compiler_params=pltpu.CompilerParams(
            dimension_semantics=("parallel", "arbitrary"),
            vmem_limit_bytes=_VMEM,
        ),
        cost_estimate=pl.CostEstimate(
            flops=flops, transcendentals=0, bytes_accessed=bytes_accessed),
    )(jnp.asarray(i_map), jnp.asarray(j_map), jnp.asarray(b_map),
      jnp.asarray(act), a_hat, p)


def _xw1_kernel(x_ref, w1_ref, p_ref):
    p_ref[...] = jnp.dot(
        x_ref[...].astype(jnp.bfloat16), w1_ref[...],
        preferred_element_type=jnp.float32,
    ).astype(p_ref.dtype)


def _combine1_kernel(acc_ref, b1_ref, w2_ref, q_ref):
    s = acc_ref[0] + acc_ref[1]
    h = jnp.maximum(s + b1_ref[...], 0.0)
    q_ref[...] = jnp.dot(h.astype(jnp.bfloat16), w2_ref[...],
                         preferred_element_type=jnp.float32).astype(q_ref.dtype)


def _combine2_kernel(acc_ref, b2_ref, out_ref):
    s = acc_ref[0] + acc_ref[1]
    out_ref[...] = jnp.maximum(s + b2_ref[...], 0.0)


@jax.jit
def _gcn(a_hat, in_feat, w1, b1, w2, b2):
    n, f = in_feat.shape
    h = w1.shape[1]
    c = w2.shape[1]
    tm = 1024

    w1_b = w1.astype(jnp.bfloat16)
    w2_b = w2.astype(jnp.bfloat16)
    b1_f = b1.astype(jnp.float32).reshape(1, h)
    b2_f = b2.astype(jnp.float32).reshape(1, c)

    cparams = pltpu.CompilerParams(
        dimension_semantics=("parallel",), vmem_limit_bytes=_VMEM)

    tp = 1024
    p = pl.pallas_call(
        _xw1_kernel,
        grid=(n // tp,),
        out_shape=jax.ShapeDtypeStruct((n, h), jnp.bfloat16),
        in_specs=[pl.BlockSpec((tp, f), lambda i: (i, 0)),
                  pl.BlockSpec((f, h), lambda i: (0, 0))],
        out_specs=pl.BlockSpec((tp, h), lambda i: (i, 0)),
        compiler_params=cparams,
        cost_estimate=pl.CostEstimate(
            flops=2 * n * f * h, transcendentals=0,
            bytes_accessed=4 * n * f + 2 * (f * h + n * h)),
    )(in_feat, w1_b)

    acc1 = _sym_matmul(a_hat, p, tm)
    tr = 1024
    q = pl.pallas_call(
        _combine1_kernel,
        grid=(n // tr,),
        out_shape=jax.ShapeDtypeStruct((n, c), jnp.bfloat16),
        in_specs=[pl.BlockSpec((2, tr, h), lambda i: (0, i, 0)),
                  pl.BlockSpec((1, h), lambda i: (0, 0)),
                  pl.BlockSpec((h, c), lambda i: (0, 0))],
        out_specs=pl.BlockSpec((tr, c), lambda i: (i, 0)),
        compiler_params=cparams,
        cost_estimate=pl.CostEstimate(
            flops=2 * n * h * c, transcendentals=0,
            bytes_accessed=8 * n * h + 2 * (h * c + n * c)),
    )(acc1, b1_f, w2_b)

    acc2 = _sym_matmul(a_hat, q, tm)
    out = pl.pallas_call(
        _combine2_kernel,
        grid=(n // tr,),
        out_shape=jax.ShapeDtypeStruct((n, c), jnp.float32),
        in_specs=[pl.BlockSpec((2, tr, c), lambda i: (0, i, 0)),
                  pl.BlockSpec((1, c), lambda i: (0, 0))],
        out_specs=pl.BlockSpec((tr, c), lambda i: (i, 0)),
        compiler_params=cparams,
        cost_estimate=pl.CostEstimate(
            flops=n * c, transcendentals=0,
            bytes_accessed=8 * n * c + 4 * (c + n * c)),
    )(acc2, b2_f)
    return out


def kernel(a_hat, in_feat, w1, b1, w2, b2):
    return _gcn(a_hat, in_feat, w1, b1, w2, b2)


# speedup vs baseline: 1.1351x; 1.1351x over previous
import functools

import jax
import jax.numpy as jnp
import numpy as np
from jax.experimental import pallas as pl
from jax.experimental.pallas import tpu as pltpu

_VMEM = 64 * 1024 * 1024


def _triangle_schedule(t):
    blocks = [(i, j) for i in range(t) for j in range(i + 1)]
    core0 = blocks[0::2]
    core1 = blocks[1::2]
    m = max(len(core0), len(core1))

    def pack(lst):
        act = [1] * len(lst) + [0] * (m - len(lst))
        filler = lst[-1] if lst else (0, 0)
        lst = lst + [filler] * (m - len(lst))
        return lst, act

    core0, act0 = pack(core0)
    core1, act1 = pack(core1)
    i_map = np.array([[b[0] for b in core0], [b[0] for b in core1]], np.int32)
    j_map = np.array([[b[1] for b in core0], [b[1] for b in core1]], np.int32)
    b_map = np.array([[b[0] * (b[0] + 1) // 2 + b[1] for b in core0],
                      [b[0] * (b[0] + 1) // 2 + b[1] for b in core1]],
                     np.int32)
    act = np.array([act0, act1], np.int32)
    return i_map, j_map, b_map, act, m


def _tri_dots(acc_ref, pk_ref, p_ref, i, j, active, tm):
    cm = min(tm, 1024)

    @pl.when(active == 1)
    def _do():
        pj = p_ref[pl.ds(j * tm, tm), :]
        for r in range(tm // cm):
            acc_ref[pl.ds(i * tm + r * cm, cm), :] += jax.lax.dot_general(
                pk_ref[0, pl.ds(r * cm, cm), :], pj, (((1,), (0,)), ((), ())),
                preferred_element_type=jnp.float32)

        @pl.when(i != j)
        def _sym():
            pi = p_ref[pl.ds(i * tm, tm), :]
            for s in range(tm // cm):
                acc_ref[pl.ds(j * tm + s * cm, cm), :] += jax.lax.dot_general(
                    pk_ref[0, :, pl.ds(s * cm, cm)], pi,
                    (((0,), (0,)), ((), ())),
                    preferred_element_type=jnp.float32)


def _sym_acc_kernel(i_ref, j_ref, b_ref, act_ref, a_ref, p_ref, out_ref,
                    a_bf_ref, acc_ref, sem, *, tm):
    c = pl.program_id(0)
    m = pl.program_id(1)
    nm = pl.num_programs(1)
    i = i_ref[c, m]
    j = j_ref[c, m]
    active = act_ref[c, m]

    @pl.when(m == 0)
    def _zero():
        acc_ref[...] = jnp.zeros_like(acc_ref)

    a_bf_ref[0] = a_ref[...].astype(jnp.bfloat16)
    _tri_dots(acc_ref, a_bf_ref, p_ref, i, j, active, tm)

    @pl.when(m == nm - 1)
    def _flush():
        copy = pltpu.make_async_copy(acc_ref, out_ref.at[c], sem)
        copy.start()
        copy.wait()


def _sym_matmul(a_hat, p, tm):
    n = a_hat.shape[0]
    k = p.shape[1]
    t = n // tm
    i_map, j_map, b_map, act, m = _triangle_schedule(t)
    tri_blocks = t * (t + 1) // 2

    grid_spec = pltpu.PrefetchScalarGridSpec(
        num_scalar_prefetch=4,
        grid=(2, m),
        in_specs=[
            pl.BlockSpec((tm, tm),
                         lambda c, mm, i_m, j_m, b_m, a_m: (i_m[c, mm],
                                                            j_m[c, mm])),
            pl.BlockSpec((n, k), lambda c, mm, i_m, j_m, b_m, a_m: (0, 0)),
        ],
        out_specs=pl.BlockSpec(memory_space=pl.ANY),
        scratch_shapes=[pltpu.VMEM((1, tm, tm), jnp.bfloat16),
                        pltpu.VMEM((n, k), jnp.float32),
                        pltpu.SemaphoreType.DMA],
    )
    flops = 2 * (2 * tri_blocks - t) * tm * tm * k
    bytes_accessed = 4 * tri_blocks * tm * tm + 2 * n * k + 2 * 4 * n * k
    return pl.pallas_call(
        functools.partial(_sym_acc_kernel, tm=tm),
        grid_spec=grid_spec,
        out_shape=jax.ShapeDtypeStruct((2, n, k), jnp.float32),
        compiler_params=pltpu.CompilerParams(
            dimension_semantics=("parallel", "arbitrary"),
            vmem_limit_bytes=_VMEM,
        ),
        cost_estimate=pl.CostEstimate(
            flops=flops, transcendentals=0, bytes_accessed=bytes_accessed),
    )(jnp.asarray(i_map), jnp.asarray(j_map), jnp.asarray(b_map),
      jnp.asarray(act), a_hat, p)


def _xw1_kernel(x_ref, w1_ref, p_ref):
    p_ref[...] = jnp.dot(
        x_ref[...].astype(jnp.bfloat16), w1_ref[...],
        preferred_element_type=jnp.float32,
    ).astype(p_ref.dtype)


def _combine1_kernel(acc_ref, b1_ref, w2_ref, q_ref):
    s = acc_ref[0] + acc_ref[1]
    h = jnp.maximum(s + b1_ref[...], 0.0)
    q_ref[...] = jnp.dot(h.astype(jnp.bfloat16), w2_ref[...],
                         preferred_element_type=jnp.float32).astype(q_ref.dtype)


def _combine2_kernel(acc_ref, b2_ref, out_ref):
    s = acc_ref[0] + acc_ref[1]
    out_ref[...] = jnp.maximum(s + b2_ref[...], 0.0)


@jax.jit
def _gcn(a_hat, in_feat, w1, b1, w2, b2):
    n, f = in_feat.shape
    h = w1.shape[1]
    c = w2.shape[1]
    tm = 2048

    w1_b = w1.astype(jnp.bfloat16)
    w2_b = w2.astype(jnp.bfloat16)
    b1_f = b1.astype(jnp.float32).reshape(1, h)
    b2_f = b2.astype(jnp.float32).reshape(1, c)

    cparams = pltpu.CompilerParams(
        dimension_semantics=("parallel",), vmem_limit_bytes=_VMEM)

    tp = 1024
    p = pl.pallas_call(
        _xw1_kernel,
        grid=(n // tp,),
        out_shape=jax.ShapeDtypeStruct((n, h), jnp.bfloat16),
        in_specs=[pl.BlockSpec((tp, f), lambda i: (i, 0)),
                  pl.BlockSpec((f, h), lambda i: (0, 0))],
        out_specs=pl.BlockSpec((tp, h), lambda i: (i, 0)),
        compiler_params=cparams,
        cost_estimate=pl.CostEstimate(
            flops=2 * n * f * h, transcendentals=0,
            bytes_accessed=4 * n * f + 2 * (f * h + n * h)),
    )(in_feat, w1_b)

    acc1 = _sym_matmul(a_hat, p, tm)
    tr = 1024
    q = pl.pallas_call(
        _combine1_kernel,
        grid=(n // tr,),
        out_shape=jax.ShapeDtypeStruct((n, c), jnp.bfloat16),
        in_specs=[pl.BlockSpec((2, tr, h), lambda i: (0, i, 0)),
                  pl.BlockSpec((1, h), lambda i: (0, 0)),
                  pl.BlockSpec((h, c), lambda i: (0, 0))],
        out_specs=pl.BlockSpec((tr, c), lambda i: (i, 0)),
        compiler_params=cparams,
        cost_estimate=pl.CostEstimate(
            flops=2 * n * h * c, transcendentals=0,
            bytes_accessed=8 * n * h + 2 * (h * c + n * c)),
    )(acc1, b1_f, w2_b)

    acc2 = _sym_matmul(a_hat, q, tm)
    out = pl.pallas_call(
        _combine2_kernel,
        grid=(n // tr,),
        out_shape=jax.ShapeDtypeStruct((n, c), jnp.float32),
        in_specs=[pl.BlockSpec((2, tr, c), lambda i: (0, i, 0)),
                  pl.BlockSpec((1, c), lambda i: (0, 0))],
        out_specs=pl.BlockSpec((tr, c), lambda i: (i, 0)),
        compiler_params=cparams,
        cost_estimate=pl.CostEstimate(
            flops=n * c, transcendentals=0,
            bytes_accessed=8 * n * c + 4 * (c + n * c)),
    )(acc2, b2_f)
    return out


def kernel(a_hat, in_feat, w1, b1, w2, b2):
    return _gcn(a_hat, in_feat, w1, b1, w2, b2)


# speedup vs baseline: 1.1722x; 1.0326x over previous
import functools

import jax
import jax.numpy as jnp
import numpy as np
from jax.experimental import pallas as pl
from jax.experimental.pallas import tpu as pltpu

_VMEM = 64 * 1024 * 1024


def _triangle_schedule(t):
    blocks = [(i, j) for i in range(t) for j in range(i + 1)]
    core0 = blocks[0::2]
    core1 = blocks[1::2]
    m = max(len(core0), len(core1))

    def pack(lst):
        act = [1] * len(lst) + [0] * (m - len(lst))
        filler = lst[-1] if lst else (0, 0)
        lst = lst + [filler] * (m - len(lst))
        return lst, act

    core0, act0 = pack(core0)
    core1, act1 = pack(core1)
    i_map = np.array([[b[0] for b in core0], [b[0] for b in core1]], np.int32)
    j_map = np.array([[b[1] for b in core0], [b[1] for b in core1]], np.int32)
    b_map = np.array([[b[0] * (b[0] + 1) // 2 + b[1] for b in core0],
                      [b[0] * (b[0] + 1) // 2 + b[1] for b in core1]],
                     np.int32)
    act = np.array([act0, act1], np.int32)
    return i_map, j_map, b_map, act, m


def _tri_dots(acc_ref, pk_ref, p_ref, i, j, active, tm):
    cm = min(tm, 1024)

    @pl.when(active == 1)
    def _do():
        pj = p_ref[pl.ds(j * tm, tm), :]
        for r in range(tm // cm):
            acc_ref[pl.ds(i * tm + r * cm, cm), :] += jax.lax.dot_general(
                pk_ref[0, pl.ds(r * cm, cm), :], pj, (((1,), (0,)), ((), ())),
                preferred_element_type=jnp.float32)

        @pl.when(i != j)
        def _sym():
            pi = p_ref[pl.ds(i * tm, tm), :]
            for s in range(tm // cm):
                acc_ref[pl.ds(j * tm + s * cm, cm), :] += jax.lax.dot_general(
                    pk_ref[0, :, pl.ds(s * cm, cm)], pi,
                    (((0,), (0,)), ((), ())),
                    preferred_element_type=jnp.float32)


def _sym_acc_kernel(i_ref, j_ref, b_ref, act_ref, a_ref, p_ref, out_ref,
                    a_bf_ref, acc_ref, lo_ref, sem, *, tm):
    c = pl.program_id(0)
    m = pl.program_id(1)
    nm = pl.num_programs(1)
    i = i_ref[c, m]
    j = j_ref[c, m]
    active = act_ref[c, m]

    @pl.when(m == 0)
    def _zero():
        acc_ref[...] = jnp.zeros_like(acc_ref)

    a_bf_ref[0] = a_ref[...].astype(jnp.bfloat16)
    _tri_dots(acc_ref, a_bf_ref, p_ref, i, j, active, tm)

    @pl.when(m == nm - 1)
    def _flush():
        lo_ref[...] = acc_ref[...].astype(jnp.bfloat16)
        copy = pltpu.make_async_copy(lo_ref, out_ref.at[c], sem)
        copy.start()
        copy.wait()


def _sym_matmul(a_hat, p, tm):
    n = a_hat.shape[0]
    k = p.shape[1]
    t = n // tm
    i_map, j_map, b_map, act, m = _triangle_schedule(t)
    tri_blocks = t * (t + 1) // 2

    grid_spec = pltpu.PrefetchScalarGridSpec(
        num_scalar_prefetch=4,
        grid=(2, m),
        in_specs=[
            pl.BlockSpec((tm, tm),
                         lambda c, mm, i_m, j_m, b_m, a_m: (i_m[c, mm],
                                                            j_m[c, mm])),
            pl.BlockSpec((n, k), lambda c, mm, i_m, j_m, b_m, a_m: (0, 0)),
        ],
        out_specs=pl.BlockSpec(memory_space=pl.ANY),
        scratch_shapes=[pltpu.VMEM((1, tm, tm), jnp.bfloat16),
                        pltpu.VMEM((n, k), jnp.float32),
                        pltpu.VMEM((n, k), jnp.bfloat16),
                        pltpu.SemaphoreType.DMA],
    )
    flops = 2 * (2 * tri_blocks - t) * tm * tm * k
    bytes_accessed = 4 * tri_blocks * tm * tm + 2 * n * k + 2 * 4 * n * k
    return pl.pallas_call(
        functools.partial(_sym_acc_kernel, tm=tm),
        grid_spec=grid_spec,
        out_shape=jax.ShapeDtypeStruct((2, n, k), jnp.bfloat16),
        compiler_params=pltpu.CompilerParams(
            dimension_semantics=("parallel", "arbitrary"),
            vmem_limit_bytes=_VMEM,
        ),
        cost_estimate=pl.CostEstimate(
            flops=flops, transcendentals=0, bytes_accessed=bytes_accessed),
    )(jnp.asarray(i_map), jnp.asarray(j_map), jnp.asarray(b_map),
      jnp.asarray(act), a_hat, p)


def _xw1_kernel(x_ref, w1_ref, p_ref):
    p_ref[...] = jnp.dot(
        x_ref[...].astype(jnp.bfloat16), w1_ref[...],
        preferred_element_type=jnp.float32,
    ).astype(p_ref.dtype)


def _combine1_kernel(acc_ref, b1_ref, w2_ref, q_ref):
    s = acc_ref[0].astype(jnp.float32) + acc_ref[1].astype(jnp.float32)
    h = jnp.maximum(s + b1_ref[...], 0.0)
    q_ref[...] = jnp.dot(h.astype(jnp.bfloat16), w2_ref[...],
                         preferred_element_type=jnp.float32).astype(q_ref.dtype)


def _combine2_kernel(acc_ref, b2_ref, out_ref):
    s = acc_ref[0].astype(jnp.float32) + acc_ref[1].astype(jnp.float32)
    out_ref[...] = jnp.maximum(s + b2_ref[...], 0.0)


@jax.jit
def _gcn(a_hat, in_feat, w1, b1, w2, b2):
    n, f = in_feat.shape
    h = w1.shape[1]
    c = w2.shape[1]
    tm = 2048

    w1_b = w1.astype(jnp.bfloat16)
    w2_b = w2.astype(jnp.bfloat16)
    b1_f = b1.astype(jnp.float32).reshape(1, h)
    b2_f = b2.astype(jnp.float32).reshape(1, c)

    cparams = pltpu.CompilerParams(
        dimension_semantics=("parallel",), vmem_limit_bytes=_VMEM)

    tp = 1024
    p = pl.pallas_call(
        _xw1_kernel,
        grid=(n // tp,),
        out_shape=jax.ShapeDtypeStruct((n, h), jnp.bfloat16),
        in_specs=[pl.BlockSpec((tp, f), lambda i: (i, 0)),
                  pl.BlockSpec((f, h), lambda i: (0, 0))],
        out_specs=pl.BlockSpec((tp, h), lambda i: (i, 0)),
        compiler_params=cparams,
        cost_estimate=pl.CostEstimate(
            flops=2 * n * f * h, transcendentals=0,
            bytes_accessed=4 * n * f + 2 * (f * h + n * h)),
    )(in_feat, w1_b)

    acc1 = _sym_matmul(a_hat, p, tm)
    tr = 1024
    q = pl.pallas_call(
        _combine1_kernel,
        grid=(n // tr,),
        out_shape=jax.ShapeDtypeStruct((n, c), jnp.bfloat16),
        in_specs=[pl.BlockSpec((2, tr, h), lambda i: (0, i, 0)),
                  pl.BlockSpec((1, h), lambda i: (0, 0)),
                  pl.BlockSpec((h, c), lambda i: (0, 0))],
        out_specs=pl.BlockSpec((tr, c), lambda i: (i, 0)),
        compiler_params=cparams,
        cost_estimate=pl.CostEstimate(
            flops=2 * n * h * c, transcendentals=0,
            bytes_accessed=8 * n * h + 2 * (h * c + n * c)),
    )(acc1, b1_f, w2_b)

    acc2 = _sym_matmul(a_hat, q, tm)
    out = pl.pallas_call(
        _combine2_kernel,
        grid=(n // tr,),
        out_shape=jax.ShapeDtypeStruct((n, c), jnp.float32),
        in_specs=[pl.BlockSpec((2, tr, c), lambda i: (0, i, 0)),
                  pl.BlockSpec((1, c), lambda i: (0, 0))],
        out_specs=pl.BlockSpec((tr, c), lambda i: (i, 0)),
        compiler_params=cparams,
        cost_estimate=pl.CostEstimate(
            flops=n * c, transcendentals=0,
            bytes_accessed=8 * n * c + 4 * (c + n * c)),
    )(acc2, b2_f)
    return out


def kernel(a_hat, in_feat, w1, b1, w2, b2):
    return _gcn(a_hat, in_feat, w1, b1, w2, b2)
